# Initial kernel scaffold; baseline (speedup 1.0000x reference)
import jax
import jax.numpy as jnp
from jax.experimental import pallas as pl


def kernel(x, text_table, pos_table):
    raise NotImplementedError("write your pallas kernel here")


# speedup vs baseline: 2.3682x; 2.3682x over previous
import functools

import jax
import jax.numpy as jnp
from jax import lax
from jax.experimental import pallas as pl
from jax.experimental.pallas import tpu as pltpu
from jax.experimental.pallas import tpu_sc as plsc

NC = 2
NS = 16
NW = NC * NS
LANES = 16


@functools.partial(jax.jit, static_argnums=(3, 4, 5, 6))
def _embed(x_flat, text_table, pos_table, out_rows, seq_len, d, seq_per_w):
    mesh = plsc.VectorSubcoreMesh(core_axis_name="c", subcore_axis_name="s")

    def body(x_hbm, text_hbm, pos_hbm, out_hbm, idx_v, rows_v, pos_v, sem):
        wid = lax.axis_index("s") * NC + lax.axis_index("c")
        base = wid * seq_per_w * seq_len

        pltpu.sync_copy(pos_hbm, pos_v)

        @pl.loop(0, seq_per_w)
        def seq_body(i):
            row0 = base + i * seq_len
            pltpu.sync_copy(x_hbm.at[pl.ds(row0, seq_len)], idx_v)
            pltpu.async_copy(text_hbm.at[idx_v], rows_v, sem).wait()

            @pl.loop(0, seq_len)
            def add_body(r):
                for cb in range(d // LANES):
                    sl = pl.ds(cb * LANES, LANES)
                    rows_v[r, sl] = rows_v[r, sl] + pos_v[r, sl]

            pltpu.sync_copy(rows_v, out_hbm.at[pl.ds(row0, seq_len)])

    total = out_rows
    fn = pl.kernel(
        body,
        out_type=jax.ShapeDtypeStruct((total, d), jnp.float32),
        mesh=mesh,
        scratch_types=[
            pltpu.VMEM((seq_len,), jnp.int32),
            pltpu.VMEM((seq_len, d), jnp.float32),
            pltpu.VMEM((seq_len, d), jnp.float32),
            pltpu.SemaphoreType.DMA,
        ],
        compiler_params=pltpu.CompilerParams(use_tc_tiling_on_sc=False),
    )
    return fn(x_flat, text_table, pos_table)


def kernel(x, text_table, pos_table):
    b, s = x.shape
    _, d = text_table.shape
    total = b * s
    seq_per_w = total // (NW * s)
    x_flat = x.reshape(total).astype(jnp.int32)
    out = _embed(x_flat, text_table, pos_table, total, s, d, seq_per_w)
    return out.reshape(b, s, d)


# speedup vs baseline: 2.8819x; 1.2169x over previous
import functools

import jax
import jax.numpy as jnp
from jax import lax
from jax.experimental import pallas as pl
from jax.experimental.pallas import tpu as pltpu
from jax.experimental.pallas import tpu_sc as plsc

NC = 2
NS = 16
NW = NC * NS
LANES = 16
PADW = 129


@functools.partial(jax.jit, static_argnums=(3, 4, 5))
def _embed(xt_flat, text_table, pos_table, b, s_len, d):
    bpw = b // NW
    n_dh = d // 8
    ngrp = d // LANES
    mesh = plsc.VectorSubcoreMesh(core_axis_name="c", subcore_axis_name="s")

    def body(xt_hbm, text_hbm, pos_hbm, out_hbm,
             idx0, idx1, rows0, rows1, pos_v, out0, out1,
             g0, g1, o0, o1):
        w = lax.axis_index("s") * NC + lax.axis_index("c")
        pltpu.sync_copy(pos_hbm, pos_v)
        lane = lax.iota(jnp.int32, LANES)
        drow = [lane + c * LANES for c in range(ngrp)]

        idx_v = (idx0, idx1)
        rows_v = (rows0, rows1)
        out_v = (out0, out1)
        gsem = (g0, g1)
        osem = (o0, o1)

        def stage_gather(s, bsel):
            pltpu.sync_copy(xt_hbm.at[pl.ds(s * b + w * bpw, bpw)],
                            idx_v[bsel])
            pltpu.async_copy(text_hbm.at[idx_v[bsel]], rows_v[bsel],
                             gsem[bsel])

        def wait_gather(bsel):
            pltpu.make_async_copy(text_hbm.at[idx_v[bsel]], rows_v[bsel],
                                  gsem[bsel]).wait()

        def start_out(s, bsel):
            for t in range(n_dh):
                pltpu.async_copy(
                    out_v[bsel].at[pl.ds(t * 8, 8), pl.ds(0, bpw)],
                    out_hbm.at[s, t, w], osem[bsel])

        def wait_out(s, bsel):
            for t in range(n_dh):
                pltpu.make_async_copy(
                    out_v[bsel].at[pl.ds(t * 8, 8), pl.ds(0, bpw)],
                    out_hbm.at[s, t, w], osem[bsel]).wait()

        stage_gather(0, 0)

        @pl.loop(0, s_len // 2)
        def step(i):
            for b2 in (0, 1):
                s = 2 * i + b2
                nxt = 1 - b2

                @pl.when(s + 1 < s_len)
                def _():
                    stage_gather(s + 1, nxt)

                wait_gather(b2)

                @pl.when(s >= 2)
                def _():
                    wait_out(s - 2, b2)

                p = [pos_v[s, pl.ds(c * LANES, LANES)] for c in range(ngrp)]

                @pl.loop(0, bpw, unroll=4)
                def bp_body(j):
                    col = jnp.full((LANES,), j, jnp.int32)
                    for c in range(ngrp):
                        val = rows_v[b2][j, pl.ds(c * LANES, LANES)] + p[c]
                        plsc.store_scatter(out_v[b2], [drow[c], col], val)

                start_out(s, b2)

        wait_out(s_len - 2, 0)
        wait_out(s_len - 1, 1)

    fn = pl.kernel(
        body,
        out_type=jax.ShapeDtypeStruct((s_len, n_dh, NW, 8, bpw), jnp.float32),
        mesh=mesh,
        scratch_types=[
            pltpu.VMEM((bpw,), jnp.int32),
            pltpu.VMEM((bpw,), jnp.int32),
            pltpu.VMEM((bpw, d), jnp.float32),
            pltpu.VMEM((bpw, d), jnp.float32),
            pltpu.VMEM((s_len, d), jnp.float32),
            pltpu.VMEM((d, PADW), jnp.float32),
            pltpu.VMEM((d, PADW), jnp.float32),
            pltpu.SemaphoreType.DMA,
            pltpu.SemaphoreType.DMA,
            pltpu.SemaphoreType.DMA,
            pltpu.SemaphoreType.DMA,
        ],
        compiler_params=pltpu.CompilerParams(
            use_tc_tiling_on_sc=False, needs_layout_passes=False),
    )
    buf = fn(xt_flat, text_table, pos_table)
    out = buf.transpose(2, 4, 0, 1, 3)
    return out.reshape(b, s_len, d)


def kernel(x, text_table, pos_table):
    b, s = x.shape
    _, d = text_table.shape
    xt_flat = x.T.reshape(b * s).astype(jnp.int32)
    return _embed(xt_flat, text_table, pos_table, b, s, d)


# speedup vs baseline: 3.1768x; 1.1023x over previous
import functools

import jax
import jax.numpy as jnp
from jax import lax
from jax.experimental import pallas as pl
from jax.experimental.pallas import tpu as pltpu
from jax.experimental.pallas import tpu_sc as plsc

NC = 2
NS = 16
NW = NC * NS
LANES = 16
PADW = 129


@functools.partial(jax.jit, static_argnums=(3, 4, 5))
def _embed(xt, text_table, pos_table, b, s_len, d):
    bpw = b // NW
    n_dh = d // 8
    ngrp = d // LANES
    mesh = plsc.VectorSubcoreMesh(core_axis_name="c", subcore_axis_name="s")

    def body(xt_hbm, text_hbm, pos_hbm, out_hbm,
             idx_all, rows0, rows1, pos_v, out0, out1,
             g0, g1, o0, o1):
        w = lax.axis_index("s") * NC + lax.axis_index("c")
        pltpu.sync_copy(xt_hbm.at[:, pl.ds(w * bpw, bpw)], idx_all)
        pltpu.sync_copy(pos_hbm, pos_v)
        lane = lax.iota(jnp.int32, LANES)
        drow = [lane + c * LANES for c in range(ngrp)]

        rows_v = (rows0, rows1)
        out_v = (out0, out1)
        gsem = (g0, g1)
        osem = (o0, o1)

        def stage_gather(s, bsel):
            pltpu.async_copy(text_hbm.at[idx_all.at[s]], rows_v[bsel],
                             gsem[bsel])

        def wait_gather(s, bsel):
            pltpu.make_async_copy(text_hbm.at[idx_all.at[s]], rows_v[bsel],
                                  gsem[bsel]).wait()

        def start_out(s, bsel):
            for t in range(n_dh):
                pltpu.async_copy(
                    out_v[bsel].at[pl.ds(t * 8, 8), pl.ds(0, bpw)],
                    out_hbm.at[s, t, w], osem[bsel])

        def wait_out(s, bsel):
            for t in range(n_dh):
                pltpu.make_async_copy(
                    out_v[bsel].at[pl.ds(t * 8, 8), pl.ds(0, bpw)],
                    out_hbm.at[s, t, w], osem[bsel]).wait()

        stage_gather(0, 0)

        @pl.loop(0, s_len // 2)
        def step(i):
            for b2 in (0, 1):
                s = 2 * i + b2
                nxt = 1 - b2

                @pl.when(s + 1 < s_len)
                def _():
                    stage_gather(s + 1, nxt)

                wait_gather(s, b2)

                @pl.when(s >= 2)
                def _():
                    wait_out(s - 2, b2)

                p = [pos_v[s, pl.ds(c * LANES, LANES)] for c in range(ngrp)]

                @pl.loop(0, bpw, unroll=4)
                def bp_body(j):
                    col = jnp.full((LANES,), j, jnp.int32)
                    for c in range(ngrp):
                        val = rows_v[b2][j, pl.ds(c * LANES, LANES)] + p[c]
                        plsc.store_scatter(out_v[b2], [drow[c], col], val)

                start_out(s, b2)

        wait_out(s_len - 2, 0)
        wait_out(s_len - 1, 1)

    fn = pl.kernel(
        body,
        out_type=jax.ShapeDtypeStruct((s_len, n_dh, NW, 8, bpw), jnp.float32),
        mesh=mesh,
        scratch_types=[
            pltpu.VMEM((s_len, bpw), jnp.int32),
            pltpu.VMEM((bpw, d), jnp.float32),
            pltpu.VMEM((bpw, d), jnp.float32),
            pltpu.VMEM((s_len, d), jnp.float32),
            pltpu.VMEM((d, PADW), jnp.float32),
            pltpu.VMEM((d, PADW), jnp.float32),
            pltpu.SemaphoreType.DMA,
            pltpu.SemaphoreType.DMA,
            pltpu.SemaphoreType.DMA,
            pltpu.SemaphoreType.DMA,
        ],
        compiler_params=pltpu.CompilerParams(
            use_tc_tiling_on_sc=False, needs_layout_passes=False),
    )
    buf = fn(xt, text_table, pos_table)
    out = buf.transpose(2, 4, 0, 1, 3)
    return out.reshape(b, s_len, d)


def kernel(x, text_table, pos_table):
    b, s = x.shape
    _, d = text_table.shape
    xt = x.T.astype(jnp.int32)
    return _embed(xt, text_table, pos_table, b, s, d)
